# baseline (speedup 1.0000x reference)
import functools

import jax
import jax.numpy as jnp
from jax import lax
from jax.experimental import pallas as pl
from jax.experimental.pallas import tpu as pltpu
from jax.experimental.pallas import tpu_sc as plsc

N = 2048
NW = 32
SRC_PER_W = N // NW
NVREG = N // 16
TB = 8
NTB = NVREG // TB

_BIG = 3.0e38


def _sc_body(sx_hbm, sy_hbm, sz_hbm, tx_hbm, ty_hbm, tz_hbm,
             rowmin_hbm, colmin_hbm,
             sx_v, sy_v, sz_v, tx_v, ty_v, tz_v, tt_v,
             sxb_v, syb_v, szb_v, ssb_v,
             rowmin_v, colmin_v):
    wid = lax.axis_index("c") * 16 + lax.axis_index("s")
    base = wid * SRC_PER_W

    pltpu.sync_copy(sx_hbm.at[pl.ds(base, SRC_PER_W)], sx_v)
    pltpu.sync_copy(sy_hbm.at[pl.ds(base, SRC_PER_W)], sy_v)
    pltpu.sync_copy(sz_hbm.at[pl.ds(base, SRC_PER_W)], sz_v)
    pltpu.sync_copy(tx_hbm, tx_v)
    pltpu.sync_copy(ty_hbm, ty_v)
    pltpu.sync_copy(tz_hbm, tz_v)

    def _b16(v):
        return v.astype(jnp.bfloat16).astype(jnp.float32)

    for g in range(SRC_PER_W // 16):
        sxg = sx_v[pl.ds(g * 16, 16)]
        syg = sy_v[pl.ds(g * 16, 16)]
        szg = sz_v[pl.ds(g * 16, 16)]
        for lane in range(16):
            s = g * 16 + lane
            sxo = jnp.full((16,), sxg[lane], jnp.float32)
            syo = jnp.full((16,), syg[lane], jnp.float32)
            szo = jnp.full((16,), szg[lane], jnp.float32)
            ssb_v[pl.ds(s * 16, 16)] = sxo * sxo + syo * syo + szo * szo
            sxb_v[pl.ds(s * 16, 16)] = _b16(sxo) * -2.0
            syb_v[pl.ds(s * 16, 16)] = _b16(syo) * -2.0
            szb_v[pl.ds(s * 16, 16)] = _b16(szo) * -2.0

    def _tt_body(i, c):
        x = tx_v[pl.ds(i * 16, 16)]
        y = ty_v[pl.ds(i * 16, 16)]
        z = tz_v[pl.ds(i * 16, 16)]
        tt_v[pl.ds(i * 16, 16)] = x * x + y * y + z * z
        return c

    lax.fori_loop(0, NVREG, _tt_body, 0)

    for tb in range(NTB):
        txb = [_b16(tx_v[pl.ds((tb * TB + k) * 16, 16)]) for k in range(TB)]
        tyb = [_b16(ty_v[pl.ds((tb * TB + k) * 16, 16)]) for k in range(TB)]
        tzb = [_b16(tz_v[pl.ds((tb * TB + k) * 16, 16)]) for k in range(TB)]
        ttb = [tt_v[pl.ds((tb * TB + k) * 16, 16)] for k in range(TB)]

        def _src_body(s, cms, txb=txb, tyb=tyb, tzb=tzb, ttb=ttb, tb=tb):
            sxb = sxb_v[pl.ds(s * 16, 16)]
            syb = syb_v[pl.ds(s * 16, 16)]
            szb = szb_v[pl.ds(s * 16, 16)]
            ssf = ssb_v[pl.ds(s * 16, 16)]
            if tb == 0:
                rm = jnp.full((16,), _BIG, jnp.float32)
            else:
                rm = rowmin_v[pl.ds(s * 16, 16)]
            new_cms = []
            for k in range(TB):
                ndot = sxb * txb[k] + syb * tyb[k] + szb * tzb[k]
                dist = (ttb[k] + ndot) + ssf
                rm = jnp.minimum(rm, dist)
                new_cms.append(jnp.minimum(cms[k], dist))
            rowmin_v[pl.ds(s * 16, 16)] = rm
            return tuple(new_cms)

        cms0 = tuple(jnp.full((16,), _BIG, jnp.float32) for _ in range(TB))
        cms = lax.fori_loop(0, SRC_PER_W, _src_body, cms0)
        for k in range(TB):
            colmin_v[pl.ds((tb * TB + k) * 16, 16)] = cms[k]

    pltpu.sync_copy(rowmin_v, rowmin_hbm.at[wid])
    pltpu.sync_copy(colmin_v, colmin_hbm.at[wid])


def _combine_body(rowmin_ref, colmin_ref, out_ref):
    rowsum = jnp.sum(jnp.min(rowmin_ref[...], axis=1))
    cmin = jnp.min(colmin_ref[...], axis=0, keepdims=True)
    loss_s2t = rowsum / N
    loss_t2s = jnp.sum(cmin) / N
    out_ref[0, 0] = loss_s2t + 0.8 * loss_t2s


def kernel(source_cloud, target_cloud):
    sx = source_cloud[:, 0]
    sy = source_cloud[:, 1]
    sz = source_cloud[:, 2]
    tx = target_cloud[:, 0]
    ty = target_cloud[:, 1]
    tz = target_cloud[:, 2]

    mesh = plsc.VectorSubcoreMesh(core_axis_name="c", subcore_axis_name="s")
    f32 = jnp.float32
    sc = functools.partial(
        pl.kernel, mesh=mesh,
        out_type=[
            jax.ShapeDtypeStruct((NW, SRC_PER_W * 16), f32),
            jax.ShapeDtypeStruct((NW, N), f32),
        ],
        scratch_types=[
            pltpu.VMEM((SRC_PER_W,), f32),
            pltpu.VMEM((SRC_PER_W,), f32),
            pltpu.VMEM((SRC_PER_W,), f32),
            pltpu.VMEM((N,), f32),
            pltpu.VMEM((N,), f32),
            pltpu.VMEM((N,), f32),
            pltpu.VMEM((N,), f32),
            pltpu.VMEM((SRC_PER_W * 16,), f32),
            pltpu.VMEM((SRC_PER_W * 16,), f32),
            pltpu.VMEM((SRC_PER_W * 16,), f32),
            pltpu.VMEM((SRC_PER_W * 16,), f32),
            pltpu.VMEM((SRC_PER_W * 16,), f32),
            pltpu.VMEM((N,), f32),
        ],
    )(_sc_body)
    rowmin_part, colmin_part = sc(sx, sy, sz, tx, ty, tz)

    out = pl.pallas_call(
        _combine_body,
        out_specs=pl.BlockSpec(memory_space=pltpu.SMEM),
        out_shape=jax.ShapeDtypeStruct((1, 1), f32),
    )(rowmin_part.reshape(N, 16), colmin_part)
    return out[0, 0]


# speedup vs baseline: 4.8098x; 4.8098x over previous
import jax
import jax.numpy as jnp
from jax.experimental import pallas as pl
from jax.experimental.pallas import tpu as pltpu

N = 2048


def _body(src_ref, tgt_ref, out_ref):
    src = src_ref[...]
    tgt = tgt_ref[...]
    ones = jnp.ones((1, 3), dtype=jnp.float32)
    tt = jax.lax.dot_general(
        ones, tgt * tgt, (((1,), (1,)), ((), ())),
        preferred_element_type=jnp.float32,
        precision=jax.lax.Precision.HIGHEST,
    )
    ndot = jax.lax.dot_general(
        src * -2.0, tgt, (((1,), (1,)), ((), ())),
        preferred_element_type=jnp.float32,
        precision=jax.lax.Precision.DEFAULT,
    )
    ss = jnp.sum(src * src, axis=1, keepdims=True)
    dist = (tt + ndot) + ss
    loss_s2t = jnp.sum(jnp.min(dist, axis=1)) / N
    loss_t2s = jnp.sum(jnp.min(dist, axis=0)) / N
    out_ref[0, 0] = loss_s2t + 0.8 * loss_t2s


def kernel(source_cloud, target_cloud):
    out = pl.pallas_call(
        _body,
        out_specs=pl.BlockSpec(memory_space=pltpu.SMEM),
        out_shape=jax.ShapeDtypeStruct((1, 1), jnp.float32),
    )(source_cloud, target_cloud)
    return out[0, 0]


# speedup vs baseline: 6.4829x; 1.3479x over previous
import jax
import jax.numpy as jnp
from jax.experimental import pallas as pl
from jax.experimental.pallas import tpu as pltpu

N = 2048


def _body(src_ref, tgtT_ref, out_ref):
    src = src_ref[...]
    tgtT = tgtT_ref[...]
    tt = jnp.sum(tgtT * tgtT, axis=0, keepdims=True)
    ndot = jax.lax.dot_general(
        src * -2.0, tgtT, (((1,), (0,)), ((), ())),
        preferred_element_type=jnp.float32,
        precision=jax.lax.Precision.DEFAULT,
    )
    ss = jnp.sum(src * src, axis=1, keepdims=True)
    dist = (tt + ndot) + ss
    loss_s2t = jnp.sum(jnp.min(dist, axis=1)) / N
    loss_t2s = jnp.sum(jnp.min(dist, axis=0)) / N
    out_ref[0, 0] = loss_s2t + 0.8 * loss_t2s


def kernel(source_cloud, target_cloud):
    tgtT = target_cloud.T
    out = pl.pallas_call(
        _body,
        out_specs=pl.BlockSpec(memory_space=pltpu.SMEM),
        out_shape=jax.ShapeDtypeStruct((1, 1), jnp.float32),
    )(source_cloud, tgtT)
    return out[0, 0]


# speedup vs baseline: 6.6120x; 1.0199x over previous
import jax
import jax.numpy as jnp
from jax.experimental import pallas as pl
from jax.experimental.pallas import tpu as pltpu

N = 2048


def _body(src_ref, tgtT_ref, out_ref):
    src = src_ref[...]
    tgtT = tgtT_ref[...]
    tt = jnp.sum(tgtT * tgtT, axis=0, keepdims=True)
    ndot = jax.lax.dot_general(
        src * -2.0, tgtT, (((1,), (0,)), ((), ())),
        preferred_element_type=jnp.float32,
        precision=jax.lax.Precision.DEFAULT,
    )
    ss = jnp.sum(src * src, axis=1, keepdims=True)
    rmin = jnp.min(tt + ndot, axis=1)
    cmin = jnp.min(ndot + ss, axis=0)
    loss_s2t = (jnp.sum(rmin) + jnp.sum(ss)) / N
    loss_t2s = (jnp.sum(cmin) + jnp.sum(tt)) / N
    out_ref[0, 0] = loss_s2t + 0.8 * loss_t2s


def kernel(source_cloud, target_cloud):
    tgtT = target_cloud.T
    out = pl.pallas_call(
        _body,
        out_specs=pl.BlockSpec(memory_space=pltpu.SMEM),
        out_shape=jax.ShapeDtypeStruct((1, 1), jnp.float32),
    )(source_cloud, tgtT)
    return out[0, 0]


# speedup vs baseline: 9.6146x; 1.4541x over previous
import jax
import jax.numpy as jnp
from jax.experimental import pallas as pl
from jax.experimental.pallas import tpu as pltpu

N = 2048


def _body(srcT_ref, tgtT_ref, out_ref):
    srcT = srcT_ref[...]
    tgtT = tgtT_ref[...]
    tt = jnp.sum(tgtT * tgtT, axis=0, keepdims=True)
    ndot = jax.lax.dot_general(
        srcT * -2.0, tgtT, (((0,), (0,)), ((), ())),
        preferred_element_type=jnp.float32,
        precision=jax.lax.Precision.DEFAULT,
    )
    ss = jnp.sum(srcT * srcT, axis=0, keepdims=True)
    rmin = jnp.min(tt + ndot, axis=1)
    cmin = jnp.min(ndot + ss.T, axis=0)
    loss_s2t = (jnp.sum(rmin) + jnp.sum(ss)) / N
    loss_t2s = (jnp.sum(cmin) + jnp.sum(tt)) / N
    out_ref[0, 0] = loss_s2t + 0.8 * loss_t2s


def kernel(source_cloud, target_cloud):
    srcT = source_cloud.T
    tgtT = target_cloud.T
    out = pl.pallas_call(
        _body,
        out_specs=pl.BlockSpec(memory_space=pltpu.SMEM),
        out_shape=jax.ShapeDtypeStruct((1, 1), jnp.float32),
    )(srcT, tgtT)
    return out[0, 0]
